# Initial kernel scaffold; baseline (speedup 1.0000x reference)
import jax
import jax.numpy as jnp
from jax.experimental import pallas as pl


def kernel(x, edge_index, batch, W1, b1, W2, b2, W3, b3, t2_W1, t2_b1, t2_W2, t2_b2, c_W1, c_b1, c_W2, c_b2):
    raise NotImplementedError("write your pallas kernel here")


# speedup vs baseline: 2.7288x; 2.7288x over previous
import functools

import jax
import jax.numpy as jnp
from jax import lax
from jax.experimental import pallas as pl
from jax.experimental.pallas import tpu as pltpu
from jax.experimental.pallas import tpu_sc as plsc

N = 50000
E = 800000
DIN = 16
DH = 128
G = 64
NC = 2
NS = 16
NW = NC * NS
NP = 51200
SPT = NP // NW
CHUNK = 1024
NCH = 25
EPT = CHUNK * NCH
E_PAD = EPT * NW
NQ = 10
NR = NP // NQ
TR = 1024
ACC_R = 6272
ZR = ACC_R // NS
WR = NR // NS // 2
ECH = 512
EPT2 = E_PAD // NS
NCH2 = EPT2 // ECH
BN = 2048
NB = NP // BN

_f32 = jnp.float32
_i32 = jnp.int32

_MESH = plsc.VectorSubcoreMesh(core_axis_name="c", subcore_axis_name="s")
_HIGH = jax.lax.Precision.HIGHEST


def _dot(a, b):
    return jax.lax.dot_general(a, b, (((1,), (0,)), ((), ())),
                               precision=_HIGH,
                               preferred_element_type=_f32)


@functools.partial(
    pl.kernel,
    out_type=jax.ShapeDtypeStruct((NC * NP,), _f32),
    mesh=_MESH,
    scratch_types=[
        pltpu.VMEM((CHUNK,), _i32),
        pltpu.VMEM((CHUNK,), _f32),
        pltpu.VMEM((SPT,), _f32),
        pltpu.VMEM_SHARED((NP,), _f32),
    ],
)
def _deg_kernel(dstp, ones_h, zeros_h, out, didx, ones_v, stage, acc):
    c = lax.axis_index("c")
    t = lax.axis_index("s")
    w = c * NS + t
    pltpu.sync_copy(ones_h, ones_v)
    pltpu.sync_copy(zeros_h, stage)
    pltpu.sync_copy(stage, acc.at[pl.ds(t * SPT, SPT)])
    plsc.subcore_barrier()

    def body(k, carry):
        off = w * EPT + k * CHUNK
        pltpu.sync_copy(dstp.at[pl.ds(off, CHUNK)], didx)
        pltpu.sync_copy(ones_v, acc.at[didx], add=True)
        return carry

    lax.fori_loop(0, NCH, body, 0)
    plsc.subcore_barrier()
    pltpu.sync_copy(acc.at[pl.ds(t * SPT, SPT)], stage)
    pltpu.sync_copy(stage, out.at[pl.ds(c * NP + t * SPT, SPT)])


@functools.partial(
    pl.kernel,
    out_type=jax.ShapeDtypeStruct((NP, DH), _f32),
    mesh=_MESH,
    scratch_types=[
        pltpu.VMEM((ECH,), _i32),
        pltpu.VMEM((ECH,), _i32),
        pltpu.VMEM((ECH,), _i32),
        pltpu.VMEM((ECH, DH), _f32),
        pltpu.VMEM_SHARED((ACC_R, DH), _f32),
    ],
)
def _agg_kernel(srcp, dstp, y, zeros_h, out, sidx, didx, cidx, rows, acc):
    c = lax.axis_index("c")
    t = lax.axis_index("s")
    for p in range(NQ // NC):
        base = (c * (NQ // NC) + p) * NR
        pltpu.sync_copy(zeros_h, rows.at[pl.ds(0, ZR)])
        pltpu.sync_copy(rows.at[pl.ds(0, ZR)], acc.at[pl.ds(t * ZR, ZR)])
        plsc.subcore_barrier()

        def chunk(k, carry):
            off = t * EPT2 + k * ECH
            pltpu.sync_copy(srcp.at[pl.ds(off, ECH)], sidx)
            pltpu.sync_copy(dstp.at[pl.ds(off, ECH)], didx)

            def remap(q, carry2):
                d = didx[pl.ds(q * 16, 16)]
                local = d - base
                ok = (local >= 0) & (local < NR)
                cidx[pl.ds(q * 16, 16)] = jnp.where(
                    ok, local, NR + (d & (TR - 1)))
                return carry2

            lax.fori_loop(0, ECH // 16, remap, 0)
            pltpu.sync_copy(y.at[sidx], rows)
            pltpu.sync_copy(rows, acc.at[cidx], add=True)
            return carry

        lax.fori_loop(0, NCH2, chunk, 0)
        plsc.subcore_barrier()
        for j in range(2):
            pltpu.sync_copy(acc.at[pl.ds(t * (2 * WR) + j * WR, WR)],
                            rows.at[pl.ds(0, WR)])
            pltpu.sync_copy(rows.at[pl.ds(0, WR)],
                            out.at[pl.ds(base + t * (2 * WR) + j * WR, WR)])


def _tc1_body(deg2_ref, x_ref, w1_ref, dinv_ref, yfull_ref):
    d = deg2_ref[0] + deg2_ref[1] + 1.0
    dv = jax.lax.rsqrt(d)
    dinv_ref[...] = dv
    yfull_ref[...] = _dot(x_ref[...], w1_ref[...]) * dv


def _tc23_body(s_ref, y_ref, dinv_ref, b_ref, w_ref, yfull_ref):
    dv = dinv_ref[...]
    h = jnp.maximum((s_ref[...] + y_ref[...]) * dv + b_ref[...], 0.0)
    yfull_ref[...] = _dot(h, w_ref[...]) * dv


def _tc4_body(s_ref, y_ref, dinv_ref, b3_ref, batch_ref,
              t2w1_ref, t2b1_ref, t2w2_ref, t2b2_ref,
              cw1_ref, cb1_ref, cw2_ref, cb2_ref,
              t2_ref, c_ref, sums_ref, cnts_ref):
    i = pl.program_id(0)
    h = jnp.maximum(
        (s_ref[...] + y_ref[...]) * dinv_ref[...] + b3_ref[...], 0.0)
    valid = (i * BN + lax.broadcasted_iota(_i32, (BN, 1), 0)) < N
    h = jnp.where(valid, h, 0.0)
    onehot = jnp.where(
        valid & (batch_ref[...] ==
                 lax.broadcasted_iota(_i32, (BN, G), 1)), 1.0, 0.0)
    ps = jax.lax.dot_general(onehot, h, (((0,), (0,)), ((), ())),
                             precision=_HIGH, preferred_element_type=_f32)
    pc = jax.lax.dot_general(onehot, jnp.ones((BN, DH), _f32),
                             (((0,), (0,)), ((), ())),
                             precision=_HIGH, preferred_element_type=_f32)

    @pl.when(i == 0)
    def _():
        sums_ref[...] = ps
        cnts_ref[...] = pc

    @pl.when(i > 0)
    def _():
        sums_ref[...] += ps
        cnts_ref[...] += pc

    @pl.when(i == NB - 1)
    def _():
        pooled = sums_ref[...] / jnp.maximum(cnts_ref[...], 1.0)
        t2h = jnp.maximum(_dot(pooled, t2w1_ref[...]) + t2b1_ref[...], 0.0)
        t2_ref[...] = _dot(t2h, t2w2_ref[...]) + t2b2_ref[...]
        ch = jnp.maximum(_dot(pooled, cw1_ref[...]) + cb1_ref[...], 0.0)
        c_ref[...] = jax.nn.sigmoid(_dot(ch, cw2_ref[...]) + cb2_ref[...])


def _row_spec(cols):
    return pl.BlockSpec((BN, cols), lambda i: (i, 0))


def _full_spec(shape):
    nd = len(shape)
    return pl.BlockSpec(shape, lambda i: (0,) * nd)


def _tc23(s, y, dinv, b, w):
    return pl.pallas_call(
        _tc23_body,
        grid=(NB,),
        in_specs=[_row_spec(DH), _row_spec(DH), _row_spec(1),
                  _full_spec((1, DH)), _full_spec((DH, DH))],
        out_specs=_row_spec(DH),
        out_shape=jax.ShapeDtypeStruct((NP, DH), _f32),
    )(s, y, dinv, b.reshape(1, DH), w)


def kernel(x, edge_index, batch, W1, b1, W2, b2, W3, b3,
           t2_W1, t2_b1, t2_W2, t2_b2, c_W1, c_b1, c_W2, c_b2):
    src = edge_index[0].astype(_i32)
    dst = edge_index[1].astype(_i32)

    pad = E_PAD - E
    ar = jnp.arange(pad, dtype=_i32)
    srcp = jnp.concatenate([src, ar % (N - 1)])
    dstp = jnp.concatenate([dst, N + (ar % (NP - N))])

    ones_c = jnp.ones((CHUNK,), _f32)
    zeros_1 = jnp.zeros((SPT,), _f32)
    zeros_z = jnp.zeros((ZR, DH), _f32)

    deg2 = _deg_kernel(dstp, ones_c, zeros_1).reshape(NC, NP, 1)

    dinv, y1 = pl.pallas_call(
        _tc1_body,
        grid=(NB,),
        in_specs=[pl.BlockSpec((NC, BN, 1), lambda i: (0, i, 0)),
                  _row_spec(DIN), _full_spec((DIN, DH))],
        out_specs=[_row_spec(1), _row_spec(DH)],
        out_shape=[jax.ShapeDtypeStruct((NP, 1), _f32),
                   jax.ShapeDtypeStruct((NP, DH), _f32)],
    )(deg2, x, W1)

    s1 = _agg_kernel(srcp, dstp, y1, zeros_z)
    y2 = _tc23(s1, y1, dinv, b1, W2)
    s2 = _agg_kernel(srcp, dstp, y2, zeros_z)
    y3 = _tc23(s2, y2, dinv, b2, W3)
    s3 = _agg_kernel(srcp, dstp, y3, zeros_z)

    t2, c = pl.pallas_call(
        _tc4_body,
        grid=(NB,),
        in_specs=[_row_spec(DH), _row_spec(DH), _row_spec(1),
                  _full_spec((1, DH)), _row_spec(1),
                  _full_spec((DH, G)), _full_spec((1, G)),
                  _full_spec((G, 1)), _full_spec((1, 1)),
                  _full_spec((DH, G)), _full_spec((1, G)),
                  _full_spec((G, 1)), _full_spec((1, 1))],
        out_specs=[_full_spec((G, 1)), _full_spec((G, 1))],
        out_shape=[jax.ShapeDtypeStruct((G, 1), _f32),
                   jax.ShapeDtypeStruct((G, 1), _f32)],
        scratch_shapes=[pltpu.VMEM((G, DH), _f32),
                        pltpu.VMEM((G, DH), _f32)],
    )(s3, y3, dinv, b3.reshape(1, DH), batch.reshape(N, 1),
      t2_W1, t2_b1.reshape(1, G), t2_W2, t2_b2.reshape(1, 1),
      c_W1, c_b1.reshape(1, G), c_W2, c_b2.reshape(1, 1))
    return (t2, c)


# speedup vs baseline: 2.9921x; 1.0965x over previous
import functools

import jax
import jax.numpy as jnp
from jax import lax
from jax.experimental import pallas as pl
from jax.experimental.pallas import tpu as pltpu
from jax.experimental.pallas import tpu_sc as plsc

N = 50000
E = 800000
DIN = 16
DH = 128
G = 64
NC = 2
NS = 16
NW = NC * NS
NP = 51200
SPT = NP // NW
CHUNK = 1024
NCH = 25
EPT = CHUNK * NCH
E_PAD = EPT * NW
NQ = 10
NR = NP // NQ
TR = 1024
ACC_R = 6272
ZR = ACC_R // NS
WR = NR // NS // 2
ECH = 256
EPT2 = E_PAD // NS
NCH2 = EPT2 // (2 * ECH)
BN = 2048
NB = NP // BN

_f32 = jnp.float32
_i32 = jnp.int32

_MESH = plsc.VectorSubcoreMesh(core_axis_name="c", subcore_axis_name="s")
_HIGH = jax.lax.Precision.HIGHEST


def _dot(a, b):
    return jax.lax.dot_general(a, b, (((1,), (0,)), ((), ())),
                               precision=_HIGH,
                               preferred_element_type=_f32)


@functools.partial(
    pl.kernel,
    out_type=jax.ShapeDtypeStruct((NC * NP,), _f32),
    mesh=_MESH,
    scratch_types=[
        pltpu.VMEM((CHUNK,), _i32),
        pltpu.VMEM((CHUNK,), _f32),
        pltpu.VMEM((SPT,), _f32),
        pltpu.VMEM_SHARED((NP,), _f32),
    ],
)
def _deg_kernel(dstp, ones_h, zeros_h, out, didx, ones_v, stage, acc):
    c = lax.axis_index("c")
    t = lax.axis_index("s")
    w = c * NS + t
    pltpu.sync_copy(ones_h, ones_v)
    pltpu.sync_copy(zeros_h, stage)
    pltpu.sync_copy(stage, acc.at[pl.ds(t * SPT, SPT)])
    plsc.subcore_barrier()

    def body(k, carry):
        off = w * EPT + k * CHUNK
        pltpu.sync_copy(dstp.at[pl.ds(off, CHUNK)], didx)
        pltpu.sync_copy(ones_v, acc.at[didx], add=True)
        return carry

    lax.fori_loop(0, NCH, body, 0)
    plsc.subcore_barrier()
    pltpu.sync_copy(acc.at[pl.ds(t * SPT, SPT)], stage)
    pltpu.sync_copy(stage, out.at[pl.ds(c * NP + t * SPT, SPT)])


@functools.partial(
    pl.kernel,
    out_type=jax.ShapeDtypeStruct((NP, DH), _f32),
    mesh=_MESH,
    scratch_types=[
        pltpu.VMEM((ECH,), _i32),
        pltpu.VMEM((ECH,), _i32),
        pltpu.VMEM((ECH,), _i32),
        pltpu.VMEM((ECH,), _i32),
        pltpu.VMEM((ECH,), _i32),
        pltpu.VMEM((ECH,), _i32),
        pltpu.VMEM((ECH, DH), _f32),
        pltpu.VMEM((ECH, DH), _f32),
        pltpu.SemaphoreType.DMA,
        pltpu.SemaphoreType.DMA,
        pltpu.VMEM_SHARED((ACC_R, DH), _f32),
    ],
)
def _agg_kernel(srcp, dstp, y, zeros_h, out, sidx0, didx0, cidx0,
                sidx1, didx1, cidx1, rows0, rows1, gsem, ssem, acc):
    c = lax.axis_index("c")
    t = lax.axis_index("s")

    def load_remap(off, sidx, didx, cidx, base):
        pltpu.sync_copy(srcp.at[pl.ds(off, ECH)], sidx)
        pltpu.sync_copy(dstp.at[pl.ds(off, ECH)], didx)

        def remap(q, carry2):
            d = didx[pl.ds(q * 16, 16)]
            local = d - base
            ok = (local >= 0) & (local < NR)
            cidx[pl.ds(q * 16, 16)] = jnp.where(
                ok, local, NR + (d & (TR - 1)))
            return carry2

        lax.fori_loop(0, ECH // 16, remap, 0)

    for p in range(NQ // NC):
        base = (c * (NQ // NC) + p) * NR
        for zo, zn in ((0, 128), (128, 128), (256, 120)):
            pltpu.sync_copy(zeros_h.at[pl.ds(0, zn)],
                            rows0.at[pl.ds(0, zn)])
            pltpu.sync_copy(rows0.at[pl.ds(0, zn)],
                            acc.at[pl.ds(t * ZR + zo, zn)])
        plsc.subcore_barrier()

        def pair(k, carry):
            off = t * EPT2 + 2 * k * ECH
            load_remap(off, sidx0, didx0, cidx0, base)
            g0 = pltpu.make_async_copy(y.at[sidx0], rows0, gsem)
            g0.start()
            load_remap(off + ECH, sidx1, didx1, cidx1, base)
            g0.wait()
            s0 = pltpu.make_async_copy(rows0, acc.at[cidx0], ssem)
            s0.start(add=True)
            g1 = pltpu.make_async_copy(y.at[sidx1], rows1, gsem)
            g1.start()
            g1.wait()
            s0.wait()
            s1 = pltpu.make_async_copy(rows1, acc.at[cidx1], ssem)
            s1.start(add=True)
            s1.wait()
            return carry

        lax.fori_loop(0, NCH2, pair, 0)
        plsc.subcore_barrier()
        for j in range(2):
            pltpu.sync_copy(acc.at[pl.ds(t * (2 * WR) + j * WR, WR)],
                            rows0.at[pl.ds(0, WR)])
            pltpu.sync_copy(rows0.at[pl.ds(0, WR)],
                            out.at[pl.ds(base + t * (2 * WR) + j * WR, WR)])
        plsc.subcore_barrier()


def _tc1_body(deg2_ref, x_ref, w1_ref, dinv_ref, yfull_ref):
    d = deg2_ref[0] + deg2_ref[1] + 1.0
    dv = jax.lax.rsqrt(d)
    dinv_ref[...] = dv
    yfull_ref[...] = _dot(x_ref[...], w1_ref[...]) * dv


def _tc23_body(s_ref, y_ref, dinv_ref, b_ref, w_ref, yfull_ref):
    dv = dinv_ref[...]
    h = jnp.maximum((s_ref[...] + y_ref[...]) * dv + b_ref[...], 0.0)
    yfull_ref[...] = _dot(h, w_ref[...]) * dv


def _tc4_body(s_ref, y_ref, dinv_ref, b3_ref, batch_ref,
              t2w1_ref, t2b1_ref, t2w2_ref, t2b2_ref,
              cw1_ref, cb1_ref, cw2_ref, cb2_ref,
              t2_ref, c_ref, sums_ref, cnts_ref):
    i = pl.program_id(0)
    h = jnp.maximum(
        (s_ref[...] + y_ref[...]) * dinv_ref[...] + b3_ref[...], 0.0)
    valid = (i * BN + lax.broadcasted_iota(_i32, (BN, 1), 0)) < N
    h = jnp.where(valid, h, 0.0)
    onehot = jnp.where(
        valid & (batch_ref[...] ==
                 lax.broadcasted_iota(_i32, (BN, G), 1)), 1.0, 0.0)
    ps = jax.lax.dot_general(onehot, h, (((0,), (0,)), ((), ())),
                             precision=_HIGH, preferred_element_type=_f32)
    pc = jax.lax.dot_general(onehot, jnp.ones((BN, DH), _f32),
                             (((0,), (0,)), ((), ())),
                             precision=_HIGH, preferred_element_type=_f32)

    @pl.when(i == 0)
    def _():
        sums_ref[...] = ps
        cnts_ref[...] = pc

    @pl.when(i > 0)
    def _():
        sums_ref[...] += ps
        cnts_ref[...] += pc

    @pl.when(i == NB - 1)
    def _():
        pooled = sums_ref[...] / jnp.maximum(cnts_ref[...], 1.0)
        t2h = jnp.maximum(_dot(pooled, t2w1_ref[...]) + t2b1_ref[...], 0.0)
        t2_ref[...] = _dot(t2h, t2w2_ref[...]) + t2b2_ref[...]
        ch = jnp.maximum(_dot(pooled, cw1_ref[...]) + cb1_ref[...], 0.0)
        c_ref[...] = jax.nn.sigmoid(_dot(ch, cw2_ref[...]) + cb2_ref[...])


def _row_spec(cols):
    return pl.BlockSpec((BN, cols), lambda i: (i, 0))


def _full_spec(shape):
    nd = len(shape)
    return pl.BlockSpec(shape, lambda i: (0,) * nd)


def _tc23(s, y, dinv, b, w):
    return pl.pallas_call(
        _tc23_body,
        grid=(NB,),
        in_specs=[_row_spec(DH), _row_spec(DH), _row_spec(1),
                  _full_spec((1, DH)), _full_spec((DH, DH))],
        out_specs=_row_spec(DH),
        out_shape=jax.ShapeDtypeStruct((NP, DH), _f32),
    )(s, y, dinv, b.reshape(1, DH), w)


def kernel(x, edge_index, batch, W1, b1, W2, b2, W3, b3,
           t2_W1, t2_b1, t2_W2, t2_b2, c_W1, c_b1, c_W2, c_b2):
    src = edge_index[0].astype(_i32)
    dst = edge_index[1].astype(_i32)

    pad = E_PAD - E
    ar = jnp.arange(pad, dtype=_i32)
    srcp = jnp.concatenate([src, ar % (N - 1)])
    dstp = jnp.concatenate([dst, N + (ar % (NP - N))])

    ones_c = jnp.ones((CHUNK,), _f32)
    zeros_1 = jnp.zeros((SPT,), _f32)
    zeros_z = jnp.zeros((ZR, DH), _f32)

    deg2 = _deg_kernel(dstp, ones_c, zeros_1).reshape(NC, NP, 1)

    dinv, y1 = pl.pallas_call(
        _tc1_body,
        grid=(NB,),
        in_specs=[pl.BlockSpec((NC, BN, 1), lambda i: (0, i, 0)),
                  _row_spec(DIN), _full_spec((DIN, DH))],
        out_specs=[_row_spec(1), _row_spec(DH)],
        out_shape=[jax.ShapeDtypeStruct((NP, 1), _f32),
                   jax.ShapeDtypeStruct((NP, DH), _f32)],
    )(deg2, x, W1)

    s1 = _agg_kernel(srcp, dstp, y1, zeros_z)
    y2 = _tc23(s1, y1, dinv, b1, W2)
    s2 = _agg_kernel(srcp, dstp, y2, zeros_z)
    y3 = _tc23(s2, y2, dinv, b2, W3)
    s3 = _agg_kernel(srcp, dstp, y3, zeros_z)

    t2, c = pl.pallas_call(
        _tc4_body,
        grid=(NB,),
        in_specs=[_row_spec(DH), _row_spec(DH), _row_spec(1),
                  _full_spec((1, DH)), _row_spec(1),
                  _full_spec((DH, G)), _full_spec((1, G)),
                  _full_spec((G, 1)), _full_spec((1, 1)),
                  _full_spec((DH, G)), _full_spec((1, G)),
                  _full_spec((G, 1)), _full_spec((1, 1))],
        out_specs=[_full_spec((G, 1)), _full_spec((G, 1))],
        out_shape=[jax.ShapeDtypeStruct((G, 1), _f32),
                   jax.ShapeDtypeStruct((G, 1), _f32)],
        scratch_shapes=[pltpu.VMEM((G, DH), _f32),
                        pltpu.VMEM((G, DH), _f32)],
    )(s3, y3, dinv, b3.reshape(1, DH), batch.reshape(N, 1),
      t2_W1, t2_b1.reshape(1, G), t2_W2, t2_b2.reshape(1, 1),
      c_W1, c_b1.reshape(1, G), c_W2, c_b2.reshape(1, 1))
    return (t2, c)
